# baseline (speedup 1.0000x reference)
import functools

import jax
import jax.numpy as jnp
from jax import lax
from jax.experimental import pallas as pl
from jax.experimental.pallas import tpu as pltpu
from jax.experimental.pallas import tpu_sc as plsc

NC = 2
NS = 16
NW = NC * NS
NBUF = 4
BB = 128
SUB = 8


@functools.lru_cache(maxsize=None)
def _build(B, L, V, D):
    assert B == NW * BB, (B, NW * BB)
    assert D % 16 == 0
    DT = D // SUB
    NQ = D // 16
    assert L % NBUF == 0
    ngrp = L // NBUF

    mesh = plsc.VectorSubcoreMesh(
        core_axis_name="c", subcore_axis_name="s",
        num_cores=NC, num_subcores=NS)

    scratch_types = (
        [pltpu.VMEM((L, D), jnp.float32),
         pltpu.VMEM((L, BB), jnp.int32)]
        + [pltpu.VMEM((BB, D), jnp.float32) for _ in range(NBUF)]
        + [pltpu.VMEM((DT, SUB, BB), jnp.float32) for _ in range(NBUF)]
        + [pltpu.SemaphoreType.DMA for _ in range(2 * NBUF)]
    )

    def body(idxt_hbm, table_hbm, pos_hbm, out_hbm, *scr):
        pos_v = scr[0]
        idxw_v = scr[1]
        g_v = scr[2:2 + NBUF]
        o_v = scr[2 + NBUF:2 + 2 * NBUF]
        s_g = scr[2 + 2 * NBUF:2 + 3 * NBUF]
        s_o = scr[2 + 3 * NBUF:2 + 4 * NBUF]

        wid = lax.axis_index("s") * NC + lax.axis_index("c")

        pltpu.sync_copy(pos_hbm, pos_v)
        pltpu.sync_copy(idxt_hbm.at[:, pl.ds(wid * BB, BB)], idxw_v)

        def start_gather(b, c):
            pltpu.async_copy(table_hbm.at[idxw_v.at[c]], g_v[b], s_g[b])

        def wait_gather(b):
            pltpu.make_async_copy(table_hbm.at[idxw_v.at[0]], g_v[b],
                                  s_g[b]).wait()

        def start_out(b, c):
            pltpu.async_copy(o_v[b], out_hbm.at[c, :, wid], s_o[b])

        def wait_out(b):
            pltpu.make_async_copy(o_v[b], out_hbm.at[0, :, 0], s_o[b]).wait()

        def compute(b, c):
            g = g_v[b]
            o = o_v[b]
            iot = lax.iota(jnp.int32, 16)
            dt_idx = [(q * 16 + iot) // SUB for q in range(NQ)]
            dd_idx = [(q * 16 + iot) % SUB for q in range(NQ)]
            pos_q = [pos_v[c, pl.ds(q * 16, 16)] for q in range(NQ)]

            def rbody(r, carry):
                bbs = jnp.full((16,), r, jnp.int32)
                for q in range(NQ):
                    x = g[r, pl.ds(q * 16, 16)] + pos_q[q]
                    plsc.store_scatter(o, [dt_idx[q], dd_idx[q], bbs], x)
                return carry

            lax.fori_loop(0, BB, rbody, 0, unroll=2)

        for b in range(NBUF):
            start_gather(b, b)

        def group(g, carry):
            for b in range(NBUF):
                c = g * NBUF + b
                wait_gather(b)

                @pl.when(g > 0)
                def _():
                    wait_out(b)

                compute(b, c)
                start_out(b, c)

                @pl.when(g < ngrp - 1)
                def _():
                    start_gather(b, c + NBUF)
            return carry

        lax.fori_loop(0, ngrp, group, 0)

        for b in range(NBUF):
            wait_out(b)

    return pl.kernel(
        body,
        out_type=jax.ShapeDtypeStruct((L, DT, B // BB, SUB, BB), jnp.float32),
        mesh=mesh,
        scratch_types=scratch_types,
        compiler_params=pltpu.CompilerParams(use_tc_tiling_on_sc=False,
                                             needs_layout_passes=False),
    )


def kernel(inputs, token_table, pos_table):
    B, L = inputs.shape
    V, D = token_table.shape
    idxt = inputs.T.astype(jnp.int32)
    a = _build(B, L, V, D)(idxt, token_table, pos_table)
    return a.transpose(2, 4, 0, 1, 3).reshape(B, L, D)


# speedup vs baseline: 1.5327x; 1.5327x over previous
import functools

import jax
import jax.numpy as jnp
from jax import lax
from jax.experimental import pallas as pl
from jax.experimental.pallas import tpu as pltpu
from jax.experimental.pallas import tpu_sc as plsc

NC = 2
NS = 16
NW = NC * NS
NBUF = 4
BB = 128
SUB = 8


@functools.lru_cache(maxsize=None)
def _build(B, L, V, D):
    assert B == NW * BB, (B, NW * BB)
    assert D % 16 == 0
    DT = D // SUB
    NQ = D // 16
    assert L % NBUF == 0
    ngrp = L // NBUF

    mesh = plsc.VectorSubcoreMesh(
        core_axis_name="c", subcore_axis_name="s",
        num_cores=NC, num_subcores=NS)

    scratch_types = (
        [pltpu.VMEM((L, D), jnp.float32),
         pltpu.VMEM((L, BB), jnp.int32)]
        + [pltpu.VMEM((BB, D), jnp.float32) for _ in range(NBUF)]
        + [pltpu.VMEM((DT, SUB, BB), jnp.float32) for _ in range(NBUF)]
        + [pltpu.SemaphoreType.DMA for _ in range(2 * NBUF)]
    )

    def body(idxt_hbm, table_hbm, pos_hbm, out_hbm, *scr):
        pos_v = scr[0]
        idxw_v = scr[1]
        g_v = scr[2:2 + NBUF]
        o_v = scr[2 + NBUF:2 + 2 * NBUF]
        s_g = scr[2 + 2 * NBUF:2 + 3 * NBUF]
        s_o = scr[2 + 3 * NBUF:2 + 4 * NBUF]

        wid = lax.axis_index("s") * NC + lax.axis_index("c")

        pltpu.sync_copy(pos_hbm, pos_v)
        pltpu.sync_copy(idxt_hbm.at[:, pl.ds(wid * BB, BB)], idxw_v)

        def start_gather(b, c):
            pltpu.async_copy(table_hbm.at[idxw_v.at[c]], g_v[b], s_g[b])

        def wait_gather(b):
            pltpu.make_async_copy(table_hbm.at[idxw_v.at[0]], g_v[b],
                                  s_g[b]).wait()

        def start_out(b, c):
            pltpu.async_copy(o_v[b], out_hbm.at[c, :, wid], s_o[b])

        def wait_out(b):
            pltpu.make_async_copy(o_v[b], out_hbm.at[0, :, 0], s_o[b]).wait()

        def compute(b, c):
            g = g_v[b]
            o = o_v[b]
            iot = lax.iota(jnp.int32, 16)
            dt_idx = [(q * 16 + iot) // SUB for q in range(NQ)]
            dd_idx = [(q * 16 + iot) % SUB for q in range(NQ)]
            pos_q = [pos_v[c, pl.ds(q * 16, 16)] for q in range(NQ)]

            @plsc.parallel_loop(0, BB, 1, unroll=4)
            def rbody(r):
                bbs = jnp.full((16,), r, jnp.int32)
                for q in range(NQ):
                    x = g[r, pl.ds(q * 16, 16)] + pos_q[q]
                    plsc.store_scatter(o, [dt_idx[q], dd_idx[q], bbs], x)

        for b in range(NBUF):
            start_gather(b, b)

        def group(g, carry):
            for b in range(NBUF):
                c = g * NBUF + b
                wait_gather(b)

                @pl.when(g > 0)
                def _():
                    wait_out(b)

                compute(b, c)
                start_out(b, c)

                @pl.when(g < ngrp - 1)
                def _():
                    start_gather(b, c + NBUF)
            return carry

        lax.fori_loop(0, ngrp, group, 0)

        for b in range(NBUF):
            wait_out(b)

    return pl.kernel(
        body,
        out_type=jax.ShapeDtypeStruct((L, DT, B // BB, SUB, BB), jnp.float32),
        mesh=mesh,
        scratch_types=scratch_types,
        compiler_params=pltpu.CompilerParams(use_tc_tiling_on_sc=False,
                                             needs_layout_passes=False),
    )


def kernel(inputs, token_table, pos_table):
    B, L = inputs.shape
    V, D = token_table.shape
    idxt = inputs.T.astype(jnp.int32)
    a = _build(B, L, V, D)(idxt, token_table, pos_table)
    return a.transpose(2, 4, 0, 1, 3).reshape(B, L, D)
